# baseline (speedup 1.0000x reference)
import jax
import jax.numpy as jnp
from jax.experimental import pallas as pl
from jax.experimental.pallas import tpu as pltpu


_U = 8


def _feature_kernel(x_ref, w_ref, h_ref):
    h_ref[...] = jnp.dot(
        x_ref[...], w_ref[...], preferred_element_type=jnp.float32
    ).astype(h_ref.dtype)


def _make_agg_kernel(half, nbits, p_h, p_o, chunk):
    mask_n = (1 << nbits) - 1

    def _agg_kernel(packed_ref, dinv_ref, h_ref, out_ref, acc_ref):
        g = pl.program_id(0)
        z = pl.program_id(1)

        @pl.when(z == 0)
        def _():
            acc_ref[...] = jnp.zeros(acc_ref.shape, acc_ref.dtype)

            def _decode(v):
                d = v & mask_n
                s = (v >> nbits) & mask_n
                w0 = v >> 30
                n = (dinv_ref[s] * dinv_ref[d]
                     * (1 - w0).astype(jnp.float32))
                slab = h_ref[pl.ds(pl.multiple_of(s * p_h, p_h), p_h), :]
                hrow = pltpu.bitcast(slab, jnp.bfloat16).astype(
                    jnp.float32) * n
                return d, hrow

            def _body(j, c):
                base = j * _U
                vs = [packed_ref[g, base + u] for u in range(_U)]
                rows = [_decode(v) for v in vs]
                for d, hrow in rows:
                    o = pl.ds(pl.multiple_of(d * p_o, p_o), p_o)
                    acc_ref[o, :] = acc_ref[o, :] + hrow
                return c

            nb_full = half // _U
            jax.lax.fori_loop(0, nb_full, _body, 0)

            def _tail(i, c):
                d, hrow = _decode(packed_ref[g, i])
                o = pl.ds(pl.multiple_of(d * p_o, p_o), p_o)
                acc_ref[o, :] = acc_ref[o, :] + hrow
                return c

            jax.lax.fori_loop(nb_full * _U, half, _tail, 0)

        out_ref[0, :, :] = acc_ref[pl.ds(z * chunk, chunk), :]

    return _agg_kernel


def kernel(x, edge_index, weight, bias):
    N, nin = x.shape
    nout = weight.shape[1]
    E = edge_index.shape[1]
    nbits = (N - 1).bit_length()
    p_h = nout // 256
    p_o = nout // 128

    e_tot = E + N
    half = (e_tot + 1) // 2
    pad = 2 * half - e_tot

    src = edge_index[0].astype(jnp.int32)
    dst = edge_index[1].astype(jnp.int32)
    keep = src != dst
    loop = jnp.arange(N, dtype=jnp.int32)
    src_a = jnp.concatenate([src, loop])
    dst_a = jnp.concatenate([dst, loop])
    ew = jnp.concatenate(
        [keep.astype(jnp.float32), jnp.ones((N,), jnp.float32)])
    ew_off = jnp.concatenate(
        [jnp.where(keep, 0, 1 << 30).astype(jnp.int32),
         jnp.zeros((N,), jnp.int32)])
    packed = (src_a << nbits) | dst_a | ew_off
    if pad:
        packed = jnp.concatenate(
            [packed, jnp.full((pad,), 1 << 30, jnp.int32)])
    packed2 = packed.reshape(2, half)

    deg = jnp.zeros((N,), jnp.float32).at[dst_a].add(ew)
    dinv = jnp.where(deg > 0, jax.lax.rsqrt(deg), 0.0)

    xb = x.astype(jnp.bfloat16)
    wb = weight.astype(jnp.bfloat16)
    bm = min(N, 1024)
    hmat = pl.pallas_call(
        _feature_kernel,
        out_shape=jax.ShapeDtypeStruct((N, nout), jnp.bfloat16),
        grid=(N // bm,),
        in_specs=[
            pl.BlockSpec((bm, nin), lambda i: (i, 0)),
            pl.BlockSpec((nin, nout), lambda i: (0, 0)),
        ],
        out_specs=pl.BlockSpec((bm, nout), lambda i: (i, 0)),
        compiler_params=pltpu.CompilerParams(
            dimension_semantics=("parallel",)),
    )(xb, wb)

    h_i32 = jax.lax.bitcast_convert_type(
        hmat.reshape(N, p_h, 2, 128).transpose(0, 1, 3, 2), jnp.int32
    ).reshape(N * p_h, 128)

    n_chunks = 16
    chunk = (N * p_o) // n_chunks
    out2 = pl.pallas_call(
        _make_agg_kernel(half, nbits, p_h, p_o, chunk),
        out_shape=jax.ShapeDtypeStruct((2, N * p_o, 128), jnp.float32),
        grid_spec=pltpu.PrefetchScalarGridSpec(
            num_scalar_prefetch=1,
            grid=(2, n_chunks),
            in_specs=[
                pl.BlockSpec(memory_space=pltpu.SMEM),
                pl.BlockSpec((N * p_h, 128), lambda g, z, packed: (0, 0)),
            ],
            out_specs=pl.BlockSpec(
                (1, chunk, 128), lambda g, z, packed: (g, z, 0)),
            scratch_shapes=[
                pltpu.VMEM((N * p_o, 128), jnp.float32)],
        ),
        compiler_params=pltpu.CompilerParams(
            dimension_semantics=("parallel", "arbitrary"),
            vmem_limit_bytes=56 * 1024 * 1024),
    )(packed2, dinv, h_i32)

    out = (out2[0] + out2[1]).reshape(N, nout) + bias[None, :].astype(
        jnp.float32)
    return out


# speedup vs baseline: 9.1769x; 9.1769x over previous
import jax
import jax.numpy as jnp
from jax.experimental import pallas as pl
from jax.experimental.pallas import tpu as pltpu


def _feature_kernel(x_ref, w_ref, h_ref):
    h_ref[...] = jnp.dot(
        x_ref[...], w_ref[...], preferred_element_type=jnp.float32
    ).astype(h_ref.dtype)


def _agg_kernel(adj_ref, h_ref, b_ref, out_ref):
    k = pl.program_id(1)

    @pl.when(k == 0)
    def _():
        out_ref[...] = jnp.broadcast_to(b_ref[...], out_ref.shape)

    out_ref[...] += jnp.dot(
        adj_ref[...], h_ref[...], preferred_element_type=jnp.float32)


def kernel(x, edge_index, weight, bias):
    N, nin = x.shape
    nout = weight.shape[1]

    src, dst = edge_index[0], edge_index[1]
    keep = (src != dst).astype(jnp.float32)
    loop = jnp.arange(N, dtype=src.dtype)
    src_a = jnp.concatenate([src, loop])
    dst_a = jnp.concatenate([dst, loop])
    ew = jnp.concatenate([keep, jnp.ones((N,), jnp.float32)])

    deg = jnp.zeros((N,), jnp.float32).at[dst_a].add(ew)
    dinv = jnp.where(deg > 0, jax.lax.rsqrt(deg), 0.0)
    dd = dinv[jnp.concatenate([src_a, dst_a])]
    e_tot = src_a.shape[0]
    norm = dd[:e_tot] * ew * dd[e_tot:]

    adj = jnp.zeros((N, N), jnp.bfloat16).at[dst_a, src_a].add(
        norm.astype(jnp.bfloat16))

    xb = x.astype(jnp.bfloat16)
    wb = weight.astype(jnp.bfloat16)
    b2 = bias.astype(jnp.float32).reshape(1, nout)

    tm, tk = 512, 2048
    hmat = pl.pallas_call(
        _feature_kernel,
        out_shape=jax.ShapeDtypeStruct((N, nout), jnp.bfloat16),
        grid=(N // 1024,),
        in_specs=[
            pl.BlockSpec((1024, nin), lambda i: (i, 0)),
            pl.BlockSpec((nin, nout), lambda i: (0, 0)),
        ],
        out_specs=pl.BlockSpec((1024, nout), lambda i: (i, 0)),
        compiler_params=pltpu.CompilerParams(
            dimension_semantics=("parallel",)),
    )(xb, wb)

    out = pl.pallas_call(
        _agg_kernel,
        out_shape=jax.ShapeDtypeStruct((N, nout), jnp.float32),
        grid=(N // tm, N // tk),
        in_specs=[
            pl.BlockSpec((tm, tk), lambda i, k: (i, k)),
            pl.BlockSpec((tk, nout), lambda i, k: (k, 0)),
            pl.BlockSpec((1, nout), lambda i, k: (0, 0)),
        ],
        out_specs=pl.BlockSpec((tm, nout), lambda i, k: (i, 0)),
        compiler_params=pltpu.CompilerParams(
            dimension_semantics=("parallel", "arbitrary")),
    )(adj, hmat, b2)

    return out


# speedup vs baseline: 12.4616x; 1.3579x over previous
import jax
import jax.numpy as jnp
from jax.experimental import pallas as pl
from jax.experimental.pallas import tpu as pltpu


def _feature_kernel(x_ref, w_ref, h_ref):
    h_ref[...] = jnp.dot(
        x_ref[...], w_ref[...], preferred_element_type=jnp.float32
    ).astype(h_ref.dtype)


def _agg_kernel(adj_ref, h_ref, b_ref, out_ref):
    k = pl.program_id(1)

    @pl.when(k == 0)
    def _():
        out_ref[...] = jnp.broadcast_to(b_ref[...], out_ref.shape)

    out_ref[...] += jnp.dot(
        adj_ref[...], h_ref[...], preferred_element_type=jnp.float32)


def kernel(x, edge_index, weight, bias):
    N, nin = x.shape
    nout = weight.shape[1]

    src, dst = edge_index[0], edge_index[1]
    keep = (src != dst).astype(jnp.float32)
    loop = jnp.arange(N, dtype=src.dtype)
    src_a = jnp.concatenate([src, loop])
    dst_a = jnp.concatenate([dst, loop])
    ew = jnp.concatenate([keep, jnp.ones((N,), jnp.float32)])

    adj0 = jnp.zeros((N, N), jnp.bfloat16).at[dst_a, src_a].add(
        ew.astype(jnp.bfloat16))
    deg = jnp.sum(adj0, axis=1, dtype=jnp.float32)
    dinv = jnp.where(deg > 0, jax.lax.rsqrt(deg), 0.0)
    adj = (dinv[:, None] * adj0.astype(jnp.float32) * dinv[None, :]
           ).astype(jnp.bfloat16)

    xb = x.astype(jnp.bfloat16)
    wb = weight.astype(jnp.bfloat16)
    b2 = bias.astype(jnp.float32).reshape(1, nout)

    tm, tk = 512, 2048
    hmat = pl.pallas_call(
        _feature_kernel,
        out_shape=jax.ShapeDtypeStruct((N, nout), jnp.bfloat16),
        grid=(N // 1024,),
        in_specs=[
            pl.BlockSpec((1024, nin), lambda i: (i, 0)),
            pl.BlockSpec((nin, nout), lambda i: (0, 0)),
        ],
        out_specs=pl.BlockSpec((1024, nout), lambda i: (i, 0)),
        compiler_params=pltpu.CompilerParams(
            dimension_semantics=("parallel",)),
    )(xb, wb)

    out = pl.pallas_call(
        _agg_kernel,
        out_shape=jax.ShapeDtypeStruct((N, nout), jnp.float32),
        grid=(N // tm, N // tk),
        in_specs=[
            pl.BlockSpec((tm, tk), lambda i, k: (i, k)),
            pl.BlockSpec((tk, nout), lambda i, k: (k, 0)),
            pl.BlockSpec((1, nout), lambda i, k: (0, 0)),
        ],
        out_specs=pl.BlockSpec((tm, nout), lambda i, k: (i, 0)),
        compiler_params=pltpu.CompilerParams(
            dimension_semantics=("parallel", "arbitrary")),
    )(adj, hmat, b2)

    return out


# speedup vs baseline: 14.7179x; 1.1811x over previous
import jax
import jax.numpy as jnp
from jax.experimental import pallas as pl
from jax.experimental.pallas import tpu as pltpu


def _feature_kernel(x_ref, w_ref, h_ref):
    h_ref[...] = jnp.dot(
        x_ref[...], w_ref[...], preferred_element_type=jnp.float32
    ).astype(h_ref.dtype)


def _agg_kernel(adj_ref, h_ref, out_ref):
    k = pl.program_id(1)

    @pl.when(k == 0)
    def _():
        out_ref[...] = jnp.zeros(out_ref.shape, out_ref.dtype)

    out_ref[...] += jnp.dot(
        adj_ref[...], h_ref[...], preferred_element_type=jnp.float32)


def kernel(x, edge_index, weight, bias):
    N, nin = x.shape
    nout = weight.shape[1]

    src, dst = edge_index[0], edge_index[1]
    keep = (src != dst).astype(jnp.float32)
    loop = jnp.arange(N, dtype=src.dtype)
    src_a = jnp.concatenate([src, loop])
    dst_a = jnp.concatenate([dst, loop])
    ew = jnp.concatenate([keep, jnp.ones((N,), jnp.float32)])

    adj = jnp.zeros((N, N), jnp.bfloat16).at[dst_a, src_a].add(
        ew.astype(jnp.bfloat16))
    deg = jnp.sum(adj, axis=1, dtype=jnp.float32)
    dinv = jnp.where(deg > 0, jax.lax.rsqrt(deg), 0.0)

    xb = x.astype(jnp.bfloat16)
    wb = weight.astype(jnp.bfloat16)
    tm, tk = 512, 2048
    hmat = pl.pallas_call(
        _feature_kernel,
        out_shape=jax.ShapeDtypeStruct((N, nout), jnp.bfloat16),
        grid=(N // 1024,),
        in_specs=[
            pl.BlockSpec((1024, nin), lambda i: (i, 0)),
            pl.BlockSpec((nin, nout), lambda i: (0, 0)),
        ],
        out_specs=pl.BlockSpec((1024, nout), lambda i: (i, 0)),
        compiler_params=pltpu.CompilerParams(
            dimension_semantics=("parallel",)),
    )(xb, wb)
    hmat = (hmat.astype(jnp.float32) * dinv[:, None]).astype(jnp.bfloat16)

    out = pl.pallas_call(
        _agg_kernel,
        out_shape=jax.ShapeDtypeStruct((N, nout), jnp.float32),
        grid=(N // tm, N // tk),
        in_specs=[
            pl.BlockSpec((tm, tk), lambda i, k: (i, k)),
            pl.BlockSpec((tk, nout), lambda i, k: (k, 0)),
        ],
        out_specs=pl.BlockSpec((tm, nout), lambda i, k: (i, 0)),
        compiler_params=pltpu.CompilerParams(
            dimension_semantics=("parallel", "arbitrary")),
    )(adj, hmat)

    return out * dinv[:, None] + bias[None, :].astype(jnp.float32)
